# baseline (speedup 1.0000x reference)
import functools

import jax
import jax.numpy as jnp
from jax import lax
from jax.experimental import pallas as pl
from jax.experimental.pallas import tpu as pltpu
from jax.experimental.pallas import tpu_sc as plsc

EMB_DIM = 64
SCALE = 8.0
NUM_CORES = 2
NUM_SUBCORES = 16
NW = NUM_CORES * NUM_SUBCORES
CHUNK = 128
LANES = 16

_MESH = dict(core_axis_name="c", subcore_axis_name="s",
             num_cores=NUM_CORES, num_subcores=NUM_SUBCORES)
_PARAMS = pltpu.CompilerParams(use_tc_tiling_on_sc=True,
                               needs_layout_passes=False)


@functools.cache
def _make_transpose(V):
    n_full = V // CHUNK
    tail = V - n_full * CHUNK
    mesh = plsc.VectorSubcoreMesh(**_MESH)

    @functools.partial(
        pl.kernel,
        out_type=jax.ShapeDtypeStruct((V, 2 * EMB_DIM), jnp.float32),
        mesh=mesh,
        scratch_types=[
            pltpu.VMEM((2, EMB_DIM, CHUNK), jnp.float32),
            pltpu.VMEM((2, CHUNK, 2 * EMB_DIM), jnp.float32),
            pltpu.VMEM((EMB_DIM, tail or 1), jnp.float32),
            pltpu.VMEM((tail or 1, 2 * EMB_DIM), jnp.float32),
            pltpu.VMEM((LANES * (LANES + 1),), jnp.float32),
            pltpu.VMEM((LANES * (LANES + 1),), jnp.float32),
            pltpu.SemaphoreType.DMA,
            pltpu.SemaphoreType.DMA,
        ],
        compiler_params=_PARAMS,
    )
    def transpose(tp_hbm, out_hbm, tin, tout, tin_t, tout_t, skew0, skew1,
                  sem_i, sem_o):
        wid = lax.axis_index("s") * NUM_CORES + lax.axis_index("c")
        k_max = (n_full - wid + NW - 1) // NW
        rows_tab = [jnp.arange(LANES, dtype=jnp.int32) + j * LANES
                    for j in range(EMB_DIM // LANES)]

        def col_of(k):
            return pl.multiple_of((wid + k * NW) * CHUNK, CHUNK)

        def start_in(k, b):
            pltpu.async_copy(
                tp_hbm.at[:, pl.ds(col_of(k), CHUNK)], tin.at[b], sem_i)

        def wait_in(k, b):
            pltpu.make_async_copy(
                tp_hbm.at[:, pl.ds(col_of(k), CHUNK)], tin.at[b], sem_i
            ).wait()

        def start_out(k, b):
            pltpu.async_copy(
                tout.at[b],
                out_hbm.at[pl.ds(col_of(k), CHUNK)],
                sem_o)

        def wait_out(k, b):
            pltpu.make_async_copy(
                tout.at[b],
                out_hbm.at[pl.ds(col_of(k), CHUNK)],
                sem_o).wait()

        iota = jnp.arange(LANES, dtype=jnp.int32)
        sidx = [iota + r * (LANES + 1) for r in range(LANES)]
        gidx = [iota * (LANES + 1) + c for c in range(LANES)]

        def compute(tin_ref, tout_ref, n_cols):
            def body(bi, _):
                col = pl.multiple_of(bi * LANES, LANES)
                for jd in range(EMB_DIM // LANES):
                    sk = skew0 if jd % 2 == 0 else skew1
                    for r in range(LANES):
                        v = tin_ref[jd * LANES + r, pl.ds(col, LANES)]
                        plsc.store_scatter(sk, [sidx[r]], v)
                    for c in range(LANES):
                        w = plsc.load_gather(sk, [gidx[c]])
                        tout_ref[col + c, pl.ds(jd * LANES, LANES)] = w
                return ()

            lax.fori_loop(0, n_cols // LANES, body, ())

        def compute_tail(tin_ref, tout_ref, n_cols):
            def body(t, _):
                cols = jnp.full((LANES,), t, jnp.int32)
                for j in range(EMB_DIM // LANES):
                    vals = plsc.load_gather(tin_ref, [rows_tab[j], cols])
                    tout_ref[t, pl.ds(j * LANES, LANES)] = vals
                return ()

            lax.fori_loop(0, n_cols, body, ())

        @pl.when(k_max > 0)
        def _():
            start_in(0, 0)

            def step(k, _):
                b = lax.rem(k, 2)

                @pl.when(k + 1 < k_max)
                def _():
                    start_in(k + 1, 1 - b)

                wait_in(k, b)

                @pl.when(k >= 2)
                def _():
                    wait_out(k - 2, b)

                compute(tin.at[b], tout.at[b], CHUNK)
                start_out(k, b)
                return ()

            lax.fori_loop(0, k_max, step, ())

            @pl.when(k_max >= 2)
            def _():
                wait_out(k_max - 2, lax.rem(k_max, 2))

            wait_out(k_max - 1, lax.rem(k_max - 1, 2))

        if tail:
            @pl.when(wid == 0)
            def _():
                pltpu.sync_copy(
                    tp_hbm.at[:, pl.ds(n_full * CHUNK, tail)], tin_t)
                compute_tail(tin_t, tout_t, tail)
                pltpu.sync_copy(
                    tout_t, out_hbm.at[pl.ds(n_full * CHUNK, tail)])

    return transpose


@functools.cache
def _make_gather(B_TOK, SEQ, V):
    n_blk = B_TOK // CHUNK
    n_total = SEQ * n_blk
    assert n_total % NW == 0
    k_max = n_total // NW
    mesh = plsc.VectorSubcoreMesh(**_MESH)

    @functools.partial(
        pl.kernel,
        out_type=jax.ShapeDtypeStruct((SEQ, EMB_DIM, B_TOK), jnp.float32),
        mesh=mesh,
        scratch_types=[
            pltpu.VMEM((2, CHUNK), jnp.int32),
            pltpu.VMEM((2, CHUNK, 2 * EMB_DIM), jnp.float32),
            pltpu.VMEM((2, EMB_DIM, CHUNK), jnp.float32),
            pltpu.VMEM((LANES * (LANES + 1),), jnp.float32),
            pltpu.VMEM((LANES * (LANES + 1),), jnp.float32),
            pltpu.SemaphoreType.DMA,
            pltpu.SemaphoreType.DMA,
            pltpu.SemaphoreType.DMA,
        ],
        compiler_params=_PARAMS,
    )
    def gather(tokt_hbm, table_hbm, out_hbm, tok_v, rows_v, tout_v,
               skew0, skew1, sem_t, sem_g, sem_o):
        wid = lax.axis_index("s") * NUM_CORES + lax.axis_index("c")
        rows_tab = [jnp.arange(LANES, dtype=jnp.int32) + j * LANES
                    for j in range(CHUNK // LANES)]

        def blk(k):
            bid = wid * k_max + k
            return bid // n_blk, bid % n_blk

        def start_tok(k, b):
            s, bb = blk(k)
            pltpu.async_copy(
                tokt_hbm.at[s, pl.ds(pl.multiple_of(bb * CHUNK, CHUNK),
                                     CHUNK)],
                tok_v.at[b], sem_t)

        def wait_tok(k, b):
            s, bb = blk(k)
            pltpu.make_async_copy(
                tokt_hbm.at[s, pl.ds(pl.multiple_of(bb * CHUNK, CHUNK),
                                     CHUNK)],
                tok_v.at[b], sem_t).wait()

        def start_gather(b):
            pltpu.async_copy(table_hbm.at[tok_v.at[b]], rows_v.at[b], sem_g)

        def wait_gather(b):
            pltpu.make_async_copy(
                table_hbm.at[tok_v.at[b]], rows_v.at[b], sem_g).wait()

        def start_out(k, b):
            s, bb = blk(k)
            pltpu.async_copy(
                tout_v.at[b],
                out_hbm.at[s, pl.ds(0, EMB_DIM),
                           pl.ds(pl.multiple_of(bb * CHUNK, CHUNK), CHUNK)],
                sem_o)

        def wait_out(k, b):
            s, bb = blk(k)
            pltpu.make_async_copy(
                tout_v.at[b],
                out_hbm.at[s, pl.ds(0, EMB_DIM),
                           pl.ds(pl.multiple_of(bb * CHUNK, CHUNK), CHUNK)],
                sem_o).wait()

        iota = jnp.arange(LANES, dtype=jnp.int32)
        sidx = [iota + r * (LANES + 1) for r in range(LANES)]
        gidx = [iota * (LANES + 1) + c for c in range(LANES)]

        def compute(b):
            def body(ji, _):
                col = pl.multiple_of(ji * LANES, LANES)
                for jd in range(EMB_DIM // LANES):
                    sk = skew0 if jd % 2 == 0 else skew1
                    for r in range(LANES):
                        v = rows_v[b, ji * LANES + r, pl.ds(jd * LANES, LANES)]
                        plsc.store_scatter(sk, [sidx[r]], v)
                    for c in range(LANES):
                        w = plsc.load_gather(sk, [gidx[c]])
                        tout_v[b, jd * LANES + c, pl.ds(col, LANES)] = (
                            w * SCALE)
                return ()

            lax.fori_loop(0, CHUNK // LANES, body, ())

        start_tok(0, 0)
        wait_tok(0, 0)
        start_gather(0)
        start_tok(1, 1)

        def step(k, _):
            b = lax.rem(k, 2)
            wait_gather(b)

            @pl.when(k + 1 < k_max)
            def _():
                wait_tok(k + 1, 1 - b)
                start_gather(1 - b)

            @pl.when(k + 2 < k_max)
            def _():
                start_tok(k + 2, b)

            @pl.when(k >= 2)
            def _():
                wait_out(k - 2, b)

            compute(b)
            start_out(k, b)
            return ()

        lax.fori_loop(0, k_max, step, ())

        @pl.when(k_max >= 2)
        def _():
            wait_out(k_max - 2, lax.rem(k_max, 2))

        wait_out(k_max - 1, lax.rem(k_max - 1, 2))

    return gather


def kernel(tokens, table):
    B_TOK, SEQ = tokens.shape
    V = table.shape[0]
    tphys = jnp.transpose(table)
    tokt = jnp.transpose(tokens).astype(jnp.int32)
    scratch = _make_transpose(V)(tphys)
    out3 = _make_gather(B_TOK, SEQ, V)(tokt, scratch)
    return jnp.transpose(out3, (2, 0, 1))


# speedup vs baseline: 1.1523x; 1.1523x over previous
import functools

import jax
import jax.numpy as jnp
from jax import lax
from jax.experimental import pallas as pl
from jax.experimental.pallas import tpu as pltpu
from jax.experimental.pallas import tpu_sc as plsc

EMB_DIM = 64
SCALE = 8.0
NUM_CORES = 2
NUM_SUBCORES = 16
NW = NUM_CORES * NUM_SUBCORES
CHUNK = 128
LANES = 16

_MESH = dict(core_axis_name="c", subcore_axis_name="s",
             num_cores=NUM_CORES, num_subcores=NUM_SUBCORES)


@functools.cache
def _make_gather(B):
    assert B % (NW * CHUNK) == 0
    b_per_w = B // NW
    g_per_w = b_per_w // CHUNK
    mesh = plsc.VectorSubcoreMesh(**_MESH)

    @functools.partial(
        pl.kernel,
        out_type=jax.ShapeDtypeStruct((B, EMB_DIM), jnp.float32),
        mesh=mesh,
        scratch_types=[
            pltpu.VMEM((g_per_w, CHUNK), jnp.int32),
            pltpu.VMEM((2, CHUNK, EMB_DIM), jnp.float32),
            pltpu.VMEM((2, CHUNK, EMB_DIM), jnp.float32),
            pltpu.SemaphoreType.DMA,
            pltpu.SemaphoreType.DMA,
        ],
        compiler_params=pltpu.CompilerParams(use_tc_tiling_on_sc=False),
    )
    def gather(tok_hbm, table_hbm, out_hbm, tok_v, rows_v, out_v,
               sem_g, sem_o):
        wid = lax.axis_index("s") * NUM_CORES + lax.axis_index("c")
        base = wid * b_per_w
        pltpu.sync_copy(tok_hbm.at[pl.ds(wid * g_per_w, g_per_w)], tok_v)

        def start_gather(g, b):
            pltpu.async_copy(table_hbm.at[tok_v.at[g]], rows_v.at[b], sem_g)

        def wait_gather(g, b):
            pltpu.make_async_copy(
                table_hbm.at[tok_v.at[g]], rows_v.at[b], sem_g).wait()

        def out_slice(g):
            return out_hbm.at[pl.ds(base + g * CHUNK, CHUNK)]

        def start_out(g, b):
            pltpu.async_copy(out_v.at[b], out_slice(g), sem_o)

        def wait_out(g, b):
            pltpu.make_async_copy(out_v.at[b], out_slice(g), sem_o).wait()

        start_gather(0, 0)

        def chunk_body(g, _):
            b = lax.rem(g, 2)
            wait_gather(g, b)

            @pl.when(g + 1 < g_per_w)
            def _():
                start_gather(g + 1, 1 - b)

            @pl.when(g >= 2)
            def _():
                wait_out(g - 2, b)

            def scale_body(r, _):
                for j in range(EMB_DIM // LANES):
                    sl = pl.ds(j * LANES, LANES)
                    out_v[b, r, sl] = rows_v[b, r, sl] * SCALE
                return ()

            lax.fori_loop(0, CHUNK, scale_body, ())
            start_out(g, b)
            return ()

        lax.fori_loop(0, g_per_w, chunk_body, ())

        @pl.when(g_per_w >= 2)
        def _():
            wait_out(g_per_w - 2, lax.rem(g_per_w, 2))

        wait_out(g_per_w - 1, lax.rem(g_per_w - 1, 2))

    return gather


def kernel(tokens, table):
    B = tokens.size
    toks = tokens.reshape(-1).astype(jnp.int32).reshape(-1, CHUNK)
    out = _make_gather(B)(toks, table)
    return out.reshape(tokens.shape + (EMB_DIM,))
